# baseline (speedup 1.0000x reference)
import jax
import jax.numpy as jnp
import numpy as np
from jax.experimental import pallas as pl
from jax.experimental.pallas import tpu as pltpu

_LANES = 128
_ROWS = 136
_M = 32
_CHUNKS = ((0, 40), (40, 32), (72, 32), (104, 32))


def _deint_matrix():
    e = np.zeros((512, 512), np.float32)
    for c in range(4):
        for l in range(128):
            e[4 * l + c, 128 * c + l] = 1.0
    return e.astype(jnp.bfloat16)


def _expand_matrices():
    e = np.zeros((5, 128, 128), np.float32)
    for t in range(5):
        for l in range(128):
            e[t, (128 * t + l) // 5, l] = 1.0
    return e.astype(jnp.bfloat16)


def _loss_body(pq_ref, rp_ref, rt_ref, cnt_ref, e_ref, et_ref, t_ref,
               thres_ref, out_ref):
    b = pl.program_id(0)
    nb = pl.num_programs(0)

    @pl.when(b == 0)
    def _():
        out_ref[...] = jnp.zeros((1, 1), jnp.float32)

    zero = jnp.float32(0.0)
    f32 = jnp.float32
    thres = thres_ref[0]
    c1 = f32(1.0) + thres

    tx1 = [t_ref[b, m, 0] for m in range(_M)]
    ty1 = [t_ref[b, m, 1] for m in range(_M)]
    tx2 = [t_ref[b, m, 2] for m in range(_M)]
    ty2 = [t_ref[b, m, 3] for m in range(_M)]
    atm = [thres * (jnp.maximum(tx2[m] - tx1[m], zero)
                    * jnp.maximum(ty2[m] - ty1[m], zero)) for m in range(_M)]

    e = e_ref[...]
    npos_acc = jnp.zeros((8, _LANES), jnp.float32)
    loss_acc = jnp.zeros((8, _LANES), jnp.float32)

    for q0, rows in _CHUNKS:
        qs = pl.ds(q0, rows)
        pq = pq_ref[qs, :]
        hi = pq.astype(jnp.bfloat16)
        r1 = pq - hi.astype(f32)
        mid = r1.astype(jnp.bfloat16)
        lo = (r1 - mid.astype(f32)).astype(jnp.bfloat16)
        coords = (jnp.dot(hi, e, preferred_element_type=jnp.float32)
                  + jnp.dot(mid, e, preferred_element_type=jnp.float32)
                  + jnp.dot(lo, e, preferred_element_type=jnp.float32))
        px1 = coords[:, 0:128]
        py1 = coords[:, 128:256]
        px2 = coords[:, 256:384]
        py2 = coords[:, 384:512]
        apt = thres * (jnp.maximum(px2 - px1, zero)
                       * jnp.maximum(py2 - py1, zero))

        hit = jnp.zeros((rows, _LANES), jnp.bool_)
        for m in range(_M):
            w = jnp.maximum(
                jnp.minimum(px2, tx2[m]) - jnp.maximum(px1, tx1[m]), zero)
            h = jnp.minimum(py2, ty2[m]) - jnp.maximum(py1, ty1[m])
            inter = w * h
            hit = jnp.logical_or(hit, inter * c1 > apt + atm[m])

        pos = jnp.where(jnp.logical_and(hit, cnt_ref[qs, :] > zero),
                        f32(1.0), zero)
        pos_b = pos.astype(jnp.bfloat16)

        for v in range(rows // 8):
            npos_acc = npos_acc + pos[8 * v:8 * v + 8, :]

        for t in range(5):
            d = (rp_ref[pl.Slice(5 * q0 + t, rows, 5), :]
                 - rt_ref[pl.Slice(5 * q0 + t, rows, 5), :])
            ad = jnp.abs(d)
            sl1 = jnp.where(ad < f32(1.0), f32(0.5) * d * d, ad - f32(0.5))
            pos_exp = jnp.dot(pos_b, et_ref[t],
                              preferred_element_type=jnp.float32)
            contrib = sl1 * pos_exp
            for v in range(rows // 8):
                loss_acc = loss_acc + contrib[8 * v:8 * v + 8, :]

    npos = jnp.sum(npos_acc)
    lsum = jnp.sum(loss_acc)
    img = jnp.where(npos > zero, lsum / npos, zero)
    out_ref[...] = out_ref[...] + (img / f32(nb)).reshape(1, 1)


def kernel(P_bbx, cls_logits, reg_preds, T_boxes, cnt_p57, reg_p57, cnt_p2,
           reg_p2, iou_thres):
    del cls_logits
    B, N, _ = P_bbx.shape
    npad = _ROWS * _LANES

    pq = jnp.pad(P_bbx.reshape(B, N * 4),
                 ((0, 0), (0, 4 * npad - 4 * N))).reshape(B, _ROWS, 512)
    rp = jnp.pad(reg_preds.reshape(B, N * 5),
                 ((0, 0), (0, 5 * npad - 5 * N))).reshape(B, 5 * _ROWS, 128)
    rt = jnp.pad(
        jnp.concatenate([reg_p2.reshape(B, -1), reg_p57.reshape(B, -1)],
                        axis=1),
        ((0, 0), (0, 5 * npad - 5 * N))).reshape(B, 5 * _ROWS, 128)
    cnt = jnp.pad(
        jnp.concatenate([cnt_p2.reshape(B, -1), cnt_p57.reshape(B, -1)],
                        axis=1),
        ((0, 0), (0, npad - N))).reshape(B, _ROWS, _LANES)
    thres = jnp.reshape(iou_thres, (1,)).astype(jnp.float32)

    out = pl.pallas_call(
        _loss_body,
        grid=(B,),
        in_specs=[
            pl.BlockSpec((None, _ROWS, 512), lambda b: (b, 0, 0)),
            pl.BlockSpec((None, 5 * _ROWS, 128), lambda b: (b, 0, 0)),
            pl.BlockSpec((None, 5 * _ROWS, 128), lambda b: (b, 0, 0)),
            pl.BlockSpec((None, _ROWS, _LANES), lambda b: (b, 0, 0)),
            pl.BlockSpec((512, 512), lambda b: (0, 0)),
            pl.BlockSpec((5, 128, 128), lambda b: (0, 0, 0)),
            pl.BlockSpec(memory_space=pltpu.SMEM),
            pl.BlockSpec(memory_space=pltpu.SMEM),
        ],
        out_specs=pl.BlockSpec((1, 1), lambda b: (0, 0)),
        out_shape=jax.ShapeDtypeStruct((1, 1), jnp.float32),
    )(pq, rp, rt, cnt, _deint_matrix(), _expand_matrices(), T_boxes, thres)
    return out


# speedup vs baseline: 1.2983x; 1.2983x over previous
import jax
import jax.numpy as jnp
import numpy as np
from jax.experimental import pallas as pl
from jax.experimental.pallas import tpu as pltpu

_LANES = 128
_ROWS = 136
_M = 32
_CHUNKS = ((0, 72), (72, 64))


def _expand_matrices():
    e = np.zeros((5, 128, 128), np.float32)
    for t in range(5):
        for l in range(128):
            e[t, (128 * t + l) // 5, l] = 1.0
    return e.astype(jnp.bfloat16)


def _loss_body(c_ref, rp_ref, rt_ref, cnt_ref, et_ref, t_ref, thres_ref,
               out_ref):
    b = pl.program_id(0)
    nb = pl.num_programs(0)

    @pl.when(b == 0)
    def _():
        out_ref[...] = jnp.zeros((1, 1), jnp.float32)

    zero = jnp.float32(0.0)
    f32 = jnp.float32
    thres = thres_ref[0]
    c1 = f32(1.0) + thres

    tx1 = [t_ref[b, m, 0] for m in range(_M)]
    ty1 = [t_ref[b, m, 1] for m in range(_M)]
    tx2 = [t_ref[b, m, 2] for m in range(_M)]
    ty2 = [t_ref[b, m, 3] for m in range(_M)]
    atm = [thres * (jnp.maximum(tx2[m] - tx1[m], zero)
                    * jnp.maximum(ty2[m] - ty1[m], zero)) for m in range(_M)]

    npos_acc = jnp.zeros((8, _LANES), jnp.float32)
    loss_acc = jnp.zeros((8, _LANES), jnp.float32)

    for q0, rows in _CHUNKS:
        qs = pl.ds(q0, rows)
        px1 = c_ref[0, qs, :]
        py1 = c_ref[1, qs, :]
        px2 = c_ref[2, qs, :]
        py2 = c_ref[3, qs, :]
        apt = thres * (jnp.maximum(px2 - px1, zero)
                       * jnp.maximum(py2 - py1, zero))

        hit = jnp.zeros((rows, _LANES), jnp.bool_)
        for m in range(_M):
            w = jnp.maximum(
                jnp.minimum(px2, tx2[m]) - jnp.maximum(px1, tx1[m]), zero)
            h = jnp.minimum(py2, ty2[m]) - jnp.maximum(py1, ty1[m])
            inter = w * h
            hit = jnp.logical_or(hit, inter * c1 > apt + atm[m])

        pos = jnp.where(jnp.logical_and(hit, cnt_ref[qs, :] > zero),
                        f32(1.0), zero)
        pos_b = pos.astype(jnp.bfloat16)

        for v in range(rows // 8):
            npos_acc = npos_acc + pos[8 * v:8 * v + 8, :]

        for t in range(5):
            d = (rp_ref[pl.Slice(5 * q0 + t, rows, 5), :]
                 - rt_ref[pl.Slice(5 * q0 + t, rows, 5), :])
            ad = jnp.abs(d)
            sl1 = jnp.where(ad < f32(1.0), f32(0.5) * d * d, ad - f32(0.5))
            pos_exp = jnp.dot(pos_b, et_ref[t],
                              preferred_element_type=jnp.float32)
            contrib = sl1 * pos_exp
            for v in range(rows // 8):
                loss_acc = loss_acc + contrib[8 * v:8 * v + 8, :]

    npos = jnp.sum(npos_acc)
    lsum = jnp.sum(loss_acc)
    img = jnp.where(npos > zero, lsum / npos, zero)
    out_ref[...] = out_ref[...] + (img / f32(nb)).reshape(1, 1)


def kernel(P_bbx, cls_logits, reg_preds, T_boxes, cnt_p57, reg_p57, cnt_p2,
           reg_p2, iou_thres):
    del cls_logits
    B, N, _ = P_bbx.shape
    npad = _ROWS * _LANES

    coords = jnp.pad(P_bbx, ((0, 0), (0, npad - N), (0, 0))).transpose(
        0, 2, 1).reshape(B, 4, _ROWS, _LANES)
    rp = jnp.pad(reg_preds.reshape(B, N * 5),
                 ((0, 0), (0, 5 * npad - 5 * N))).reshape(B, 5 * _ROWS, 128)
    rt = jnp.pad(
        jnp.concatenate([reg_p2.reshape(B, -1), reg_p57.reshape(B, -1)],
                        axis=1),
        ((0, 0), (0, 5 * npad - 5 * N))).reshape(B, 5 * _ROWS, 128)
    cnt = jnp.pad(
        jnp.concatenate([cnt_p2.reshape(B, -1), cnt_p57.reshape(B, -1)],
                        axis=1),
        ((0, 0), (0, npad - N))).reshape(B, _ROWS, _LANES)
    thres = jnp.reshape(iou_thres, (1,)).astype(jnp.float32)

    out = pl.pallas_call(
        _loss_body,
        grid=(B,),
        in_specs=[
            pl.BlockSpec((None, 4, _ROWS, _LANES), lambda b: (b, 0, 0, 0)),
            pl.BlockSpec((None, 5 * _ROWS, 128), lambda b: (b, 0, 0)),
            pl.BlockSpec((None, 5 * _ROWS, 128), lambda b: (b, 0, 0)),
            pl.BlockSpec((None, _ROWS, _LANES), lambda b: (b, 0, 0)),
            pl.BlockSpec((5, 128, 128), lambda b: (0, 0, 0)),
            pl.BlockSpec(memory_space=pltpu.SMEM),
            pl.BlockSpec(memory_space=pltpu.SMEM),
        ],
        out_specs=pl.BlockSpec((1, 1), lambda b: (0, 0)),
        out_shape=jax.ShapeDtypeStruct((1, 1), jnp.float32),
    )(coords, rp, rt, cnt, _expand_matrices(), T_boxes, thres)
    return out


# speedup vs baseline: 5.2061x; 4.0097x over previous
import jax
import jax.numpy as jnp
from jax.experimental import pallas as pl
from jax.experimental.pallas import tpu as pltpu

_LANES = 128
_ROWS = 136
_M = 32


def _loss_body(c_ref, t_ref, thres_ref, out_ref):
    b = pl.program_id(0)
    nb = pl.num_programs(0)

    @pl.when(b == 0)
    def _():
        out_ref[...] = jnp.zeros((1, 1), jnp.float32)

    zero = jnp.float32(0.0)
    thres = thres_ref[0]
    c1 = jnp.float32(1.0) + thres

    tx1 = [t_ref[b, m, 0] for m in range(_M)]
    ty1 = [t_ref[b, m, 1] for m in range(_M)]
    tx2 = [t_ref[b, m, 2] for m in range(_M)]
    ty2 = [t_ref[b, m, 3] for m in range(_M)]
    atm = [thres * (jnp.maximum(tx2[m] - tx1[m], zero)
                    * jnp.maximum(ty2[m] - ty1[m], zero)) for m in range(_M)]

    npos_acc = jnp.zeros((8, _LANES), jnp.float32)
    loss_acc = jnp.zeros((8, _LANES), jnp.float32)

    for r0, rows in ((0, 72), (72, 64)):
        sl = pl.ds(r0, rows)
        px1 = c_ref[0, sl, :]
        py1 = c_ref[1, sl, :]
        px2 = c_ref[2, sl, :]
        py2 = c_ref[3, sl, :]
        apt = thres * (jnp.maximum(px2 - px1, zero)
                       * jnp.maximum(py2 - py1, zero))

        hit = jnp.zeros((rows, _LANES), jnp.bool_)
        for m in range(_M):
            w = jnp.maximum(
                jnp.minimum(px2, tx2[m]) - jnp.maximum(px1, tx1[m]), zero)
            h = jnp.minimum(py2, ty2[m]) - jnp.maximum(py1, ty1[m])
            inter = w * h
            hit = jnp.logical_or(hit, inter * c1 > apt + atm[m])

        pos = jnp.where(jnp.logical_and(hit, c_ref[14, sl, :] > zero),
                        jnp.float32(1.0), zero)

        rowsum = jnp.zeros((rows, _LANES), jnp.float32)
        for k in range(5):
            d = c_ref[4 + k, sl, :] - c_ref[9 + k, sl, :]
            ad = jnp.abs(d)
            rowsum = rowsum + jnp.where(
                ad < jnp.float32(1.0),
                jnp.float32(0.5) * d * d, ad - jnp.float32(0.5))

        for v in range(rows // 8):
            npos_acc = npos_acc + pos[8 * v:8 * v + 8, :]
            loss_acc = loss_acc + (rowsum * pos)[8 * v:8 * v + 8, :]

    npos = jnp.sum(npos_acc)
    lsum = jnp.sum(loss_acc)
    img = jnp.where(npos > zero, lsum / npos, zero)
    out_ref[...] = out_ref[...] + (img / jnp.float32(nb)).reshape(1, 1)


def kernel(P_bbx, cls_logits, reg_preds, T_boxes, cnt_p57, reg_p57, cnt_p2,
           reg_p2, iou_thres):
    del cls_logits
    B, N, _ = P_bbx.shape
    npad = _ROWS * _LANES

    reg_t = jnp.concatenate([reg_p2.reshape(B, -1, 5), reg_p57], axis=1)
    cnt_t = jnp.concatenate([cnt_p2.reshape(B, -1, 1), cnt_p57], axis=1)
    chans = jnp.concatenate([P_bbx, reg_preds, reg_t, cnt_t], axis=2)
    chans = jnp.pad(chans, ((0, 0), (0, npad - N), (0, 0))).transpose(
        0, 2, 1).reshape(B, 15, _ROWS, _LANES)
    thres = jnp.reshape(iou_thres, (1,)).astype(jnp.float32)

    out = pl.pallas_call(
        _loss_body,
        grid=(B,),
        in_specs=[
            pl.BlockSpec((None, 15, _ROWS, _LANES), lambda b: (b, 0, 0, 0)),
            pl.BlockSpec(memory_space=pltpu.SMEM),
            pl.BlockSpec(memory_space=pltpu.SMEM),
        ],
        out_specs=pl.BlockSpec((1, 1), lambda b: (0, 0)),
        out_shape=jax.ShapeDtypeStruct((1, 1), jnp.float32),
    )(chans, T_boxes, thres)
    return out
